# baseline (speedup 1.0000x reference)
import functools

import jax
import jax.numpy as jnp
from jax import lax
from jax.experimental import pallas as pl
from jax.experimental.pallas import tpu as pltpu
from jax.experimental.pallas import tpu_sc as plsc

_K = 512
_D = 64
_B = 4096
_BLK = 2048


def _argmin_body(zt_ref, cbt_ref, idx_ref):
    zt = zt_ref[...]
    cbt = cbt_ref[...]
    cbt2 = cbt * -2.0
    st = lax.dot_general(cbt2, zt, (((0,), (0,)), ((), ())),
                         precision=lax.Precision.HIGHEST)
    cnorm = jnp.sum(cbt * cbt, axis=0)
    s = st + cnorm[:, None]
    m = jnp.min(s, axis=0, keepdims=True)
    k_iota = lax.broadcasted_iota(jnp.int32, s.shape, 0)
    idx = jnp.min(jnp.where(s <= m, k_iota, _K), axis=0)
    idx_ref[...] = idx.reshape(1, 1, _BLK)


@jax.jit
def _tc_argmin(z_t, codebook_t):
    nblk = _B // _BLK
    out = pl.pallas_call(
        _argmin_body,
        grid=(nblk,),
        in_specs=[
            pl.BlockSpec((_D, _BLK), lambda i: (0, i)),
            pl.BlockSpec((_D, _K), lambda i: (0, 0)),
        ],
        out_specs=pl.BlockSpec((1, 1, _BLK), lambda i: (i, 0, 0)),
        out_shape=jax.ShapeDtypeStruct((nblk, 1, _BLK), jnp.int32),
        compiler_params=pltpu.CompilerParams(fuse_transposed_lhs_in_matmul=True),
    )(z_t, codebook_t)
    return out.reshape(_B)


@jax.jit
def _sc_gather(codebook, indices):
    nw = 8
    b_per_w = _B // nw
    mesh = plsc.VectorSubcoreMesh(core_axis_name="c", subcore_axis_name="s",
                                  num_cores=1, num_subcores=nw)

    @functools.partial(
        pl.kernel, mesh=mesh,
        out_type=jax.ShapeDtypeStruct((_B, _D), jnp.float32),
        scratch_types=[
            pltpu.VMEM((b_per_w,), jnp.int32),
            pltpu.VMEM((b_per_w, _D), jnp.float32),
            pltpu.SemaphoreType.DMA,
        ],
        compiler_params=pltpu.CompilerParams(use_tc_tiling_on_sc=False),
    )
    def gather(cb_hbm, idx_hbm, out_hbm, idx_v, rows_v, sem):
        wid = lax.axis_index("s")
        base = wid * b_per_w
        pltpu.sync_copy(idx_hbm.at[pl.ds(base, b_per_w)], idx_v)
        pltpu.async_copy(cb_hbm.at[idx_v], rows_v, sem).wait()
        pltpu.sync_copy(rows_v, out_hbm.at[pl.ds(base, b_per_w)])

    return gather(codebook, indices)


def kernel(z_e, codebook):
    indices = _tc_argmin(z_e.T, codebook.T)
    z_q = _sc_gather(codebook, indices)
    return (z_q, indices)


# speedup vs baseline: 1.0430x; 1.0430x over previous
import functools

import jax
import jax.numpy as jnp
from jax import lax
from jax.experimental import pallas as pl
from jax.experimental.pallas import tpu as pltpu
from jax.experimental.pallas import tpu_sc as plsc

_K = 512
_D = 64
_B = 4096
_BLK = 4096


def _argmin_body(zt_ref, cbt_ref, idx_ref):
    zt = zt_ref[...]
    cbt = cbt_ref[...]
    cbt2 = cbt * -2.0
    st = lax.dot_general(cbt2, zt, (((0,), (0,)), ((), ())),
                         precision=lax.Precision.HIGHEST)
    cnorm = jnp.sum(cbt * cbt, axis=0)
    s = st + cnorm[:, None]
    m = jnp.min(s, axis=0, keepdims=True)
    k_iota = lax.broadcasted_iota(jnp.int32, s.shape, 0)
    idx = jnp.min(jnp.where(s <= m, k_iota, _K), axis=0)
    idx_ref[...] = idx.reshape(1, 1, _BLK)


@jax.jit
def _tc_argmin(z_t, codebook_t):
    nblk = _B // _BLK
    out = pl.pallas_call(
        _argmin_body,
        grid=(nblk,),
        in_specs=[
            pl.BlockSpec((_D, _BLK), lambda i: (0, i)),
            pl.BlockSpec((_D, _K), lambda i: (0, 0)),
        ],
        out_specs=pl.BlockSpec((1, 1, _BLK), lambda i: (i, 0, 0)),
        out_shape=jax.ShapeDtypeStruct((nblk, 1, _BLK), jnp.int32),
        compiler_params=pltpu.CompilerParams(fuse_transposed_lhs_in_matmul=True),
    )(z_t, codebook_t)
    return out.reshape(_B)


@jax.jit
def _sc_gather(codebook, indices):
    info = plsc.get_sparse_core_info()
    nw = info.num_subcores
    b_per_w = _B // nw
    mesh = plsc.VectorSubcoreMesh(core_axis_name="c", subcore_axis_name="s",
                                  num_cores=1)

    @functools.partial(
        pl.kernel, mesh=mesh,
        out_type=jax.ShapeDtypeStruct((_B, _D), jnp.float32),
        scratch_types=[
            pltpu.VMEM((b_per_w,), jnp.int32),
            pltpu.VMEM((b_per_w, _D), jnp.float32),
            pltpu.SemaphoreType.DMA,
        ],
        compiler_params=pltpu.CompilerParams(use_tc_tiling_on_sc=False),
    )
    def gather(cb_hbm, idx_hbm, out_hbm, idx_v, rows_v, sem):
        wid = lax.axis_index("s")
        base = wid * b_per_w
        pltpu.sync_copy(idx_hbm.at[pl.ds(base, b_per_w)], idx_v)
        pltpu.async_copy(cb_hbm.at[idx_v], rows_v, sem).wait()
        pltpu.sync_copy(rows_v, out_hbm.at[pl.ds(base, b_per_w)])

    return gather(codebook, indices)


def kernel(z_e, codebook):
    indices = _tc_argmin(z_e.T, codebook.T)
    z_q = _sc_gather(codebook, indices)
    return (z_q, indices)
